# baseline (speedup 1.0000x reference)
import functools

import jax
import jax.numpy as jnp
import numpy as np
from jax import lax
from jax.experimental import pallas as pl
from jax.experimental.pallas import tpu as pltpu
from jax.experimental.pallas import tpu_sc as plsc

_VOCAB = 100000
_D = 1024
_BATCH = 4
_SEQ = 2048
_N = _BATCH * _SEQ

_NC = 2
_NS = 16
_NW = _NC * _NS

_K = 4
_CW = _SEQ // _K
_NROWS_C = _BATCH * _CW
_BPW = _NROWS_C // _NW
_GR = 16
_NST = _BPW // _GR

_SCALE = float(np.sqrt(_D))


def _pe_table() -> np.ndarray:
    pos = np.arange(_SEQ, dtype=np.float64)[:, None]
    i = np.arange(0, _D, 2, dtype=np.float64)
    div = np.exp(-np.log(10000.0) * i / _D)
    pe = np.zeros((_SEQ, _D), dtype=np.float64)
    pe[:, 0::2] = np.sin(pos * div)
    pe[:, 1::2] = np.cos(pos * div)
    return pe.astype(np.float32)


_PE = _pe_table()


_WPB = _NW // _BATCH
_PPW = _CW // _WPB


def _sc_gather(table, tok_flat, base):
    mesh = plsc.VectorSubcoreMesh(core_axis_name="c", subcore_axis_name="s")

    @functools.partial(
        pl.kernel,
        mesh=mesh,
        out_type=jax.ShapeDtypeStruct((_NROWS_C, _D), jnp.float32),
        scratch_types=[
            pltpu.VMEM((_BPW,), jnp.int32),
        ] + [pltpu.VMEM((_GR, _D), jnp.float32) for _ in range(_NST)]
          + [pltpu.SemaphoreType.DMA for _ in range(_NST)]
          + [pltpu.SemaphoreType.DMA],
    )
    def k(table_hbm, tok_hbm, out_hbm, idx_v, *rest):
        bufs = rest[:_NST]
        gsems = rest[_NST:2 * _NST]
        wsem = rest[2 * _NST]
        wid = lax.axis_index("s") * _NC + lax.axis_index("c")
        wbase = wid * _BPW
        src = (wid // _WPB) * _SEQ + base + (wid % _WPB) * _PPW
        pltpu.sync_copy(tok_hbm.at[pl.ds(src, _BPW)], idx_v)
        for j in range(_NST):
            pltpu.async_copy(
                table_hbm.at[idx_v.at[pl.ds(j * _GR, _GR)]], bufs[j], gsems[j])
        for j in range(_NST):
            pltpu.make_async_copy(
                table_hbm.at[pl.ds(0, _GR)], bufs[j], gsems[j]).wait()
            pltpu.async_copy(
                bufs[j], out_hbm.at[pl.ds(wbase + j * _GR, _GR)], wsem)
        for j in range(_NST):
            pltpu.make_async_copy(
                table_hbm.at[pl.ds(0, _GR)], bufs[0], wsem).wait()

    return k(table, tok_flat)


def _fixup_chunk(prev, gathered, pe, c):
    def body(*refs):
        g_ref, p_ref, o_ref = refs[-3], refs[-2], refs[-1]
        o_ref[...] = g_ref[...] * _SCALE + p_ref[...]

    in_specs = [
        pl.BlockSpec((_CW, _D), lambda b: (b, 0)),
        pl.BlockSpec((_CW, _D), lambda b: (c, 0)),
    ]
    operands = [gathered, pe]
    aliases = {}
    if prev is not None:
        in_specs = [pl.BlockSpec(memory_space=pl.ANY)] + in_specs
        operands = [prev] + operands
        aliases = {0: 0}

    return pl.pallas_call(
        body,
        grid=(_BATCH,),
        in_specs=in_specs,
        out_specs=pl.BlockSpec((_CW, _D), lambda b: (b * _K + c, 0)),
        out_shape=jax.ShapeDtypeStruct((_N, _D), jnp.float32),
        input_output_aliases=aliases,
    )(*operands)


def kernel(tokens, table):
    pe = jnp.asarray(_PE)
    tok_flat = tokens.astype(jnp.int32).reshape(_N)
    gs = []
    for c in range(_K):
        gs.append(_sc_gather(table, tok_flat, c * _CW))
    out = None
    for c in range(_K):
        out = _fixup_chunk(out, gs[c], pe, c)
    return out.reshape(_BATCH, _SEQ, _D)
